# baseline (speedup 1.0000x reference)
import functools

import jax
import jax.numpy as jnp
from jax import lax
from jax.experimental import pallas as pl
from jax.experimental.pallas import tpu as pltpu
from jax.experimental.pallas import tpu_sc as plsc

VOCAB = 1000000
EMBED = 32
BATCH = 16384
HIST = 200
NC, NS, L = 2, 16, 16
NW = NC * NS
NBT = BATCH // 128
NP = HIST * NBT
PER_W = NP // NW
SLAB = 200
NSLAB = PER_W // SLAB


def _make_kernel():
    mesh = plsc.VectorSubcoreMesh(core_axis_name="c", subcore_axis_name="s")

    @functools.partial(
        pl.kernel,
        mesh=mesh,
        out_type=jax.ShapeDtypeStruct((HIST * 4 * NBT, 1024), jnp.float32),
        scratch_types=[
            pltpu.VMEM((SLAB * 128,), jnp.int32),
            pltpu.VMEM((128, EMBED), jnp.float32),
            pltpu.VMEM((128, EMBED), jnp.float32),
            pltpu.VMEM((4096,), jnp.float32),
            pltpu.SemaphoreType.DMA,
            pltpu.SemaphoreType.DMA,
        ],
        compiler_params=pltpu.CompilerParams(
            use_tc_tiling_on_sc=False, needs_layout_passes=False),
    )
    def k(idx_hbm, table_hbm, out_hbm, slab_v, gbufa, gbufb, tbuf, sema, semb):
        gbufs = (gbufa, gbufb)
        sems = (sema, semb)
        wid = lax.axis_index("s") * NC + lax.axis_index("c")
        tok0 = wid * (PER_W * 128)

        f_base = [lax.iota(jnp.int32, L) * 128 + q * (L * 128) for q in range(2)]

        def slab_load(s):
            pltpu.sync_copy(
                idx_hbm.at[pl.ds(tok0 + s * (SLAB * 128), SLAB * 128)], slab_v)

        def gather(p, buf):
            off = lax.rem(p, SLAB) * 128
            return pltpu.make_async_copy(
                table_hbm.at[slab_v.at[pl.ds(off, 128)]], gbufs[buf], sems[buf])

        slab_load(0)
        gather(0, 0).start()

        def body(p, carry):
            buf = lax.rem(p, 2)

            @pl.when(buf == 0)
            def _():
                gather(p, 0).wait()

            @pl.when(buf == 1)
            def _():
                gather(p, 1).wait()

            @pl.when(lax.rem(p, SLAB) == SLAB - 1)
            def _():
                slab_load((p + 1) // SLAB)

            @pl.when(p + 1 < PER_W)
            def _():
                @pl.when(buf == 0)
                def _():
                    gather(p + 1, 1).start()

                @pl.when(buf == 1)
                def _():
                    gather(p + 1, 0).start()

            def transpose(g):
                for bi in range(128):
                    for q in range(2):
                        row = g[bi, pl.ds(q * L, L)]
                        plsc.store_scatter(tbuf, [f_base[q] + bi], row)

            @pl.when(buf == 0)
            def _():
                transpose(gbufs[0])

            @pl.when(buf == 1)
            def _():
                transpose(gbufs[1])

            h = p // NBT
            bt = lax.rem(p, NBT)
            r = h * 512 + bt
            for fg in range(4):
                pltpu.sync_copy(tbuf.at[pl.ds(fg * 1024, 1024)],
                                out_hbm.at[r + fg * 128])
            return carry

        lax.fori_loop(0, PER_W, body, 0, unroll=False)

    return k


_kernel_call = None


def kernel(input_indices, table):
    global _kernel_call
    if _kernel_call is None:
        _kernel_call = _make_kernel()
    idx_t = input_indices.astype(jnp.int32).T.reshape(-1)
    out2 = _kernel_call(idx_t, table)
    out5 = out2.reshape(HIST, 4, NBT, 8, 128)
    return out5.transpose(2, 4, 0, 1, 3).reshape(BATCH, HIST, EMBED)


# speedup vs baseline: 1.1062x; 1.1062x over previous
import functools

import jax
import jax.numpy as jnp
from jax import lax
from jax.experimental import pallas as pl
from jax.experimental.pallas import tpu as pltpu
from jax.experimental.pallas import tpu_sc as plsc

VOCAB = 1000000
EMBED = 32
BATCH = 16384
HIST = 200
NC, NS, L = 2, 16, 16
NW = NC * NS
NBT = BATCH // 128
NP = HIST * NBT
PER_W = NP // NW
SLAB = 200
NSLAB = PER_W // SLAB
LOOKAHEAD = 3


def _make_kernel():
    mesh = plsc.VectorSubcoreMesh(core_axis_name="c", subcore_axis_name="s")

    @functools.partial(
        pl.kernel,
        mesh=mesh,
        out_type=jax.ShapeDtypeStruct((HIST * 4 * NBT, 1024), jnp.float32),
        scratch_types=[
            pltpu.VMEM((SLAB * 128,), jnp.int32),
            pltpu.VMEM((SLAB * 128,), jnp.int32),
            pltpu.VMEM((4, 128, EMBED), jnp.float32),
            pltpu.VMEM((2, 4096), jnp.float32),
            pltpu.SemaphoreType.DMA,
            pltpu.SemaphoreType.DMA,
            pltpu.SemaphoreType.DMA,
            pltpu.SemaphoreType.DMA,
            pltpu.SemaphoreType.DMA,
            pltpu.SemaphoreType.DMA,
        ],
        compiler_params=pltpu.CompilerParams(
            use_tc_tiling_on_sc=False, needs_layout_passes=False),
    )
    def k(idx_hbm, table_hbm, out_hbm, slab0, slab1, gring, tbuf,
          g0, g1, g2, g3, s0, s1):
        slabs = (slab0, slab1)
        gsems = (g0, g1, g2, g3)
        ssems = (s0, s1)
        wid = lax.axis_index("s") * NC + lax.axis_index("c")
        tok0 = wid * (PER_W * 128)

        f_base = [lax.iota(jnp.int32, L) * 128 + q * (L * 128) for q in range(2)]

        def slab_load(s, par):
            pltpu.sync_copy(
                idx_hbm.at[pl.ds(tok0 + s * (SLAB * 128), SLAB * 128)],
                slabs[par])

        def gather(p, buf, spar):
            off = lax.rem(p, SLAB) * 128
            return pltpu.make_async_copy(
                table_hbm.at[slabs[spar].at[pl.ds(off, 128)]],
                gring.at[buf], gsems[buf])

        def store(p, tpar):
            h = p // NBT
            r = h * 512 + lax.rem(p, NBT)
            return [pltpu.make_async_copy(
                        tbuf.at[tpar, pl.ds(fg * 1024, 1024)],
                        out_hbm.at[r + fg * 128], ssems[tpar])
                    for fg in range(4)]

        def transpose(g, t):
            for bi in range(128):
                for q in range(2):
                    row = g[bi, pl.ds(q * L, L)]
                    plsc.store_scatter(t, [f_base[q] + bi], row)

        slab_load(0, 0)
        for p0 in range(LOOKAHEAD):
            gather(p0, p0, 0).start()

        def body(s, carry):
            p0 = s * 4

            @pl.when(lax.rem(p0, SLAB) == 0)
            def _():
                sl = p0 // SLAB + 1

                @pl.when(sl < NSLAB)
                def _():
                    @pl.when(lax.rem(sl, 2) == 0)
                    def _():
                        slab_load(sl, 0)

                    @pl.when(lax.rem(sl, 2) == 1)
                    def _():
                        slab_load(sl, 1)

            for j in range(4):
                p = p0 + j
                jn = (j + LOOKAHEAD) % 4

                @pl.when(p + LOOKAHEAD < PER_W)
                def _():
                    pn = p + LOOKAHEAD
                    spar = lax.rem(pn // SLAB, 2)

                    @pl.when(spar == 0)
                    def _():
                        gather(pn, jn, 0).start()

                    @pl.when(spar == 1)
                    def _():
                        gather(pn, jn, 1).start()

                sparw = lax.rem(p // SLAB, 2)

                @pl.when(sparw == 0)
                def _():
                    gather(p, j, 0).wait()

                @pl.when(sparw == 1)
                def _():
                    gather(p, j, 1).wait()

                if j >= 2:
                    for d in store(p - 2, j % 2):
                        d.wait()
                else:
                    @pl.when(s > 0)
                    def _():
                        for d in store(p - 2, j % 2):
                            d.wait()

                transpose(gring.at[j], tbuf.at[j % 2])

                for d in store(p, j % 2):
                    d.start()
            return carry

        lax.fori_loop(0, PER_W // 4, body, 0, unroll=False)

        for pp in (PER_W - 2, PER_W - 1):
            for d in store(pp, pp % 2):
                d.wait()

    return k


_kernel_call = None


def kernel(input_indices, table):
    global _kernel_call
    if _kernel_call is None:
        _kernel_call = _make_kernel()
    idx_t = input_indices.astype(jnp.int32).T.reshape(-1)
    out2 = _kernel_call(idx_t, table)
    out5 = out2.reshape(HIST, 4, NBT, 8, 128)
    return out5.transpose(2, 4, 0, 1, 3).reshape(BATCH, HIST, EMBED)


# speedup vs baseline: 1.3320x; 1.2041x over previous
import functools

import jax
import jax.numpy as jnp
from jax import lax
from jax.experimental import pallas as pl
from jax.experimental.pallas import tpu as pltpu
from jax.experimental.pallas import tpu_sc as plsc

VOCAB = 1000000
EMBED = 32
BATCH = 16384
HIST = 200
NC, NS, L = 2, 16, 16
NW = NC * NS
NBT = BATCH // 128
NP = HIST * NBT
PER_W = NP // NW
SLAB = 200
NSLAB = PER_W // SLAB
LOOKAHEAD = 3


def _make_kernel():
    mesh = plsc.VectorSubcoreMesh(core_axis_name="c", subcore_axis_name="s")

    @functools.partial(
        pl.kernel,
        mesh=mesh,
        out_type=jax.ShapeDtypeStruct((HIST * 4 * NBT, 1024), jnp.float32),
        scratch_types=[
            pltpu.VMEM((SLAB * 128,), jnp.int32),
            pltpu.VMEM((SLAB * 128,), jnp.int32),
            pltpu.VMEM((4, 128, EMBED), jnp.float32),
            pltpu.VMEM((2, 4096), jnp.float32),
            pltpu.SemaphoreType.DMA,
            pltpu.SemaphoreType.DMA,
            pltpu.SemaphoreType.DMA,
            pltpu.SemaphoreType.DMA,
            pltpu.SemaphoreType.DMA,
            pltpu.SemaphoreType.DMA,
        ],
        compiler_params=pltpu.CompilerParams(
            use_tc_tiling_on_sc=False, needs_layout_passes=False),
    )
    def k(idx_hbm, table_hbm, out_hbm, slab0, slab1, gring, tbuf,
          g0, g1, g2, g3, s0, s1):
        slabs = (slab0, slab1)
        gsems = (g0, g1, g2, g3)
        ssems = (s0, s1)
        wid = lax.axis_index("s") * NC + lax.axis_index("c")
        tok0 = wid * (PER_W * 128)

        bi_vecs = [lax.iota(jnp.int32, L) + L * kk for kk in range(8)]

        def slab_load(s, par):
            pltpu.sync_copy(
                idx_hbm.at[pl.ds(tok0 + s * (SLAB * 128), SLAB * 128)],
                slabs[par])

        def gather(p, buf, spar):
            off = lax.rem(p, SLAB) * 128
            return pltpu.make_async_copy(
                table_hbm.at[slabs[spar].at[pl.ds(off, 128)]],
                gring.at[buf], gsems[buf])

        def store(p, tpar):
            h = p // NBT
            r = h * 512 + lax.rem(p, NBT)
            return [pltpu.make_async_copy(
                        tbuf.at[tpar, pl.ds(fg * 1024, 1024)],
                        out_hbm.at[r + fg * 128], ssems[tpar])
                    for fg in range(4)]

        def transpose(g, t):
            @plsc.parallel_loop(0, EMBED, 1, unroll=4)
            def _(f):
                fb = jnp.full((L,), 0, jnp.int32) + f
                for kk in range(8):
                    v = plsc.load_gather(g, [bi_vecs[kk], fb])
                    t[pl.ds(f * 128 + kk * L, L)] = v

        slab_load(0, 0)
        for p0 in range(LOOKAHEAD):
            gather(p0, p0, 0).start()

        def body(s, carry):
            p0 = s * 4

            @pl.when(lax.rem(p0, SLAB) == 0)
            def _():
                sl = p0 // SLAB + 1

                @pl.when(sl < NSLAB)
                def _():
                    @pl.when(lax.rem(sl, 2) == 0)
                    def _():
                        slab_load(sl, 0)

                    @pl.when(lax.rem(sl, 2) == 1)
                    def _():
                        slab_load(sl, 1)

            for j in range(4):
                p = p0 + j
                jn = (j + LOOKAHEAD) % 4

                @pl.when(p + LOOKAHEAD < PER_W)
                def _():
                    pn = p + LOOKAHEAD
                    spar = lax.rem(pn // SLAB, 2)

                    @pl.when(spar == 0)
                    def _():
                        gather(pn, jn, 0).start()

                    @pl.when(spar == 1)
                    def _():
                        gather(pn, jn, 1).start()

                sparw = lax.rem(p // SLAB, 2)

                @pl.when(sparw == 0)
                def _():
                    gather(p, j, 0).wait()

                @pl.when(sparw == 1)
                def _():
                    gather(p, j, 1).wait()


                transpose(gring.at[j], tbuf.at[j % 2])

                h = p // NBT
                r = h * 512 + lax.rem(p, NBT)
                for fg in range(4):
                    pltpu.sync_copy(tbuf.at[j % 2, pl.ds(fg * 1024, 1024)],
                                    out_hbm.at[r + fg * 128])
            return carry

        lax.fori_loop(0, PER_W // 4, body, 0, unroll=False)

    return k


_kernel_call = None


def kernel(input_indices, table):
    global _kernel_call
    if _kernel_call is None:
        _kernel_call = _make_kernel()
    idx_t = input_indices.astype(jnp.int32).T.reshape(-1)
    out2 = _kernel_call(idx_t, table)
    out5 = out2.reshape(HIST, 4, NBT, 8, 128)
    return out5.transpose(2, 4, 0, 1, 3).reshape(BATCH, HIST, EMBED)


# speedup vs baseline: 1.5456x; 1.1604x over previous
import functools

import jax
import jax.numpy as jnp
from jax import lax
from jax.experimental import pallas as pl
from jax.experimental.pallas import tpu as pltpu
from jax.experimental.pallas import tpu_sc as plsc

VOCAB = 1000000
EMBED = 32
BATCH = 16384
HIST = 200
NC, NS, L = 2, 16, 16
NW = NC * NS
NBT = BATCH // 128
NP = HIST * NBT
PER_W = NP // NW
SLAB = 200
NSLAB = PER_W // SLAB
LOOKAHEAD = 3


def _make_kernel():
    mesh = plsc.VectorSubcoreMesh(core_axis_name="c", subcore_axis_name="s")

    @functools.partial(
        pl.kernel,
        mesh=mesh,
        out_type=jax.ShapeDtypeStruct((HIST * 4 * NBT, 1024), jnp.float32),
        scratch_types=[
            pltpu.VMEM((SLAB * 128,), jnp.int32),
            pltpu.VMEM((SLAB * 128,), jnp.int32),
            pltpu.VMEM((4, 128, EMBED), jnp.float32),
            pltpu.VMEM((2, 4096), jnp.float32),
            pltpu.SemaphoreType.DMA,
            pltpu.SemaphoreType.DMA,
            pltpu.SemaphoreType.DMA,
            pltpu.SemaphoreType.DMA,
            pltpu.SemaphoreType.DMA,
            pltpu.SemaphoreType.DMA,
        ],
        compiler_params=pltpu.CompilerParams(
            use_tc_tiling_on_sc=False, needs_layout_passes=False),
    )
    def k(idx_hbm, table_hbm, out_hbm, slab0, slab1, gring, tbuf,
          g0, g1, g2, g3, s0, s1):
        slabs = (slab0, slab1)
        gsems = (g0, g1, g2, g3)
        ssems = (s0, s1)
        wid = lax.axis_index("s") * NC + lax.axis_index("c")
        tok0 = wid * (PER_W * 128)

        bi_vecs = [lax.iota(jnp.int32, L) + L * kk for kk in range(8)]

        def slab_load(s, par):
            pltpu.sync_copy(
                idx_hbm.at[pl.ds(tok0 + s * (SLAB * 128), SLAB * 128)],
                slabs[par])

        def gather(p, buf, spar):
            off = lax.rem(p, SLAB) * 128
            return pltpu.make_async_copy(
                table_hbm.at[slabs[spar].at[pl.ds(off, 128)]],
                gring.at[buf], gsems[buf])

        def store(p, tpar):
            h = p // NBT
            r = h * 512 + lax.rem(p, NBT)
            return [pltpu.make_async_copy(
                        tbuf.at[tpar, pl.ds(fg * 1024, 1024)],
                        out_hbm.at[r + fg * 128], ssems[tpar])
                    for fg in range(4)]

        def transpose(g, t):
            @plsc.parallel_loop(0, EMBED, 1, unroll=4)
            def _(f):
                fb = jnp.full((L,), 0, jnp.int32) + f
                for kk in range(8):
                    v = plsc.load_gather(g, [bi_vecs[kk], fb])
                    t[pl.ds(f * 128 + kk * L, L)] = v

        slab_load(0, 0)
        for p0 in range(LOOKAHEAD):
            gather(p0, p0, 0).start()

        def body(s, carry):
            p0 = s * 4

            @pl.when(lax.rem(p0, SLAB) == 0)
            def _():
                sl = p0 // SLAB + 1

                @pl.when(sl < NSLAB)
                def _():
                    @pl.when(lax.rem(sl, 2) == 0)
                    def _():
                        slab_load(sl, 0)

                    @pl.when(lax.rem(sl, 2) == 1)
                    def _():
                        slab_load(sl, 1)

            for j in range(4):
                p = p0 + j
                jn = (j + LOOKAHEAD) % 4

                @pl.when(p + LOOKAHEAD < PER_W)
                def _():
                    pn = p + LOOKAHEAD
                    spar = lax.rem(pn // SLAB, 2)

                    @pl.when(spar == 0)
                    def _():
                        gather(pn, jn, 0).start()

                    @pl.when(spar == 1)
                    def _():
                        gather(pn, jn, 1).start()

                sparw = lax.rem(p // SLAB, 2)

                @pl.when(sparw == 0)
                def _():
                    gather(p, j, 0).wait()

                @pl.when(sparw == 1)
                def _():
                    gather(p, j, 1).wait()


                if j >= 2:
                    for d in store(p - 2, j % 2):
                        d.wait()
                else:
                    @pl.when(s > 0)
                    def _():
                        for d in store(p - 2, j % 2):
                            d.wait()

                transpose(gring.at[j], tbuf.at[j % 2])

                for d in store(p, j % 2):
                    d.start()
            return carry

        lax.fori_loop(0, PER_W // 4, body, 0, unroll=False)

        for pp in (PER_W - 2, PER_W - 1):
            for d in store(pp, pp % 2):
                d.wait()

    return k


_kernel_call = None


def kernel(input_indices, table):
    global _kernel_call
    if _kernel_call is None:
        _kernel_call = _make_kernel()
    idx_t = input_indices.astype(jnp.int32).T.reshape(-1)
    out2 = _kernel_call(idx_t, table)
    out5 = out2.reshape(HIST, 4, NBT, 8, 128)
    return out5.transpose(2, 4, 0, 1, 3).reshape(BATCH, HIST, EMBED)
